# Initial kernel scaffold; baseline (speedup 1.0000x reference)
import jax
import jax.numpy as jnp
from jax.experimental import pallas as pl


def kernel(x, F_rows, F_cols, F_vals, W, b):
    raise NotImplementedError("write your pallas kernel here")


# speedup vs baseline: 6.6170x; 6.6170x over previous
import functools

import jax
import jax.numpy as jnp
from jax import lax
from jax.experimental import pallas as pl
from jax.experimental.pallas import tpu as pltpu
from jax.experimental.pallas import tpu_sc as plsc

L = 16
CH = 128
SUP = 8
NG = 4
NTEC = 16
NSC = 2


def _sc_spmm(x4f, rows2, cols4, vals2, N, n_sup):
    mesh = plsc.VectorSubcoreMesh(core_axis_name="c", subcore_axis_name="s")
    rows_per_tec = N // NTEC

    @functools.partial(
        pl.kernel,
        mesh=mesh,
        compiler_params=pltpu.CompilerParams(use_tc_tiling_on_sc=False),
        out_type=jax.ShapeDtypeStruct((NG * N, L), jnp.float32),
        scratch_types=[
            pltpu.VMEM_SHARED((N, L), jnp.float32),
            pltpu.VMEM((SUP, CH), jnp.int32),
            pltpu.VMEM((SUP, CH), jnp.int32),
            pltpu.VMEM((SUP, CH), jnp.float32),
            pltpu.VMEM((CH, L), jnp.float32),
            pltpu.VMEM((CH, L), jnp.float32),
        ],
    )
    def k(x_hbm, rows_hbm, cols_hbm, vals_hbm, out_hbm,
          acc, colsv, rowsv, valsv, gbuf, zbuf):
        c = lax.axis_index("c")
        s = lax.axis_index("s")

        def zfill(i, _):
            zbuf[i, :] = jnp.zeros((L,), jnp.float32)
            return 0
        lax.fori_loop(0, CH, zfill, 0)

        for gg in range(NG // NSC):
            g = c * (NG // NSC) + gg

            def zacc(kk, _):
                pltpu.sync_copy(zbuf, acc.at[pl.ds(s * rows_per_tec + kk * CH, CH)])
                return 0
            lax.fori_loop(0, rows_per_tec // CH, zacc, 0)
            plsc.subcore_barrier()

            def sup_body(si, _):
                r0 = (s * n_sup + si) * SUP
                pltpu.sync_copy(cols_hbm.at[pl.ds(g * (n_sup * NTEC * SUP) + r0, SUP)], colsv)
                pltpu.sync_copy(rows_hbm.at[pl.ds(r0, SUP)], rowsv)
                pltpu.sync_copy(vals_hbm.at[pl.ds(r0, SUP)], valsv)
                for j in range(SUP):
                    pltpu.sync_copy(x_hbm.at[colsv.at[j]], gbuf)

                    def scale(t16, _):
                        vv = valsv[j, pl.ds(t16 * L, L)]
                        for l in range(L):
                            t = t16 * L + l
                            gbuf[t, :] = gbuf[t, :] * vv[l]
                        return 0
                    lax.fori_loop(0, CH // L, scale, 0)
                    pltpu.sync_copy(gbuf, acc.at[rowsv.at[j]], add=True)
                return 0
            lax.fori_loop(0, n_sup, sup_body, 0)
            plsc.subcore_barrier()

            def cout(kk, _):
                off = s * rows_per_tec + kk * CH
                pltpu.sync_copy(acc.at[pl.ds(off, CH)],
                                out_hbm.at[pl.ds(g * N + off, CH)])
                return 0
            lax.fori_loop(0, rows_per_tec // CH, cout, 0)
            plsc.subcore_barrier()

    return k(x4f, rows2, cols4, vals2)


def _tc_linear(acc4, W, b2, N):
    BN = 2048

    def body(a_ref, w_ref, b_ref, o_ref):
        r = jnp.zeros((BN, 64), jnp.float32)
        for g in range(NG):
            r = r + jnp.dot(a_ref[g], w_ref[g * L:(g + 1) * L, :],
                            preferred_element_type=jnp.float32)
        o_ref[...] = r + b_ref[...]

    return pl.pallas_call(
        body,
        grid=(N // BN,),
        in_specs=[
            pl.BlockSpec((NG, BN, L), lambda i: (0, i, 0)),
            pl.BlockSpec((64, 64), lambda i: (0, 0)),
            pl.BlockSpec((1, 64), lambda i: (0, 0)),
        ],
        out_specs=pl.BlockSpec((BN, 64), lambda i: (i, 0)),
        out_shape=jax.ShapeDtypeStruct((N, 64), jnp.float32),
    )(acc4, W, b2)


def kernel(x, F_rows, F_cols, F_vals, W, b):
    N, D = x.shape
    NNZ = F_rows.shape[0]
    align = NTEC * SUP * CH
    nnz_pad = -(-NNZ // align) * align
    pad = nnz_pad - NNZ
    M = nnz_pad // CH
    n_sup = M // (NTEC * SUP)

    rows_p = jnp.pad(F_rows, (0, pad))
    cols_p = jnp.pad(F_cols, (0, pad))
    vals_p = jnp.pad(F_vals, (0, pad))

    rows2 = rows_p.reshape(M, CH)
    vals2 = vals_p.reshape(M, CH)
    cols4 = (cols_p[None, :]
             + (jnp.arange(NG, dtype=jnp.int32) * N)[:, None]).reshape(NG * M, CH)
    x4f = x.reshape(N, NG, L).transpose(1, 0, 2).reshape(NG * N, L)

    acc_flat = _sc_spmm(x4f, rows2, cols4, vals2, N, n_sup)
    acc4 = acc_flat.reshape(NG, N, L)
    return _tc_linear(acc4, W, b.reshape(1, 64), N)
